# baseline (speedup 1.0000x reference)
import functools

import jax
import jax.numpy as jnp
from jax import lax
from jax.experimental import pallas as pl
from jax.experimental.pallas import tpu as pltpu
from jax.experimental.pallas import tpu_sc as plsc

_H = 32
_D = 5
_PAIRS = _D * 3
_EROWS = 1537 * _D
_SROWS = 512
_TROWS = _EROWS + _SROWS

_NC, _NS = 2, 16
_NW = _NC * _NS
_C = 128


def _table_body(e_ref, w_ref, o_ref):
    o_ref[...] = jnp.dot(e_ref[...], w_ref[...],
                         preferred_element_type=jnp.float32) * (1.0 / 3.0)


def _build_table(edge_w, wcat):
    return pl.pallas_call(
        _table_body,
        out_shape=jax.ShapeDtypeStruct((edge_w.shape[0], _D * _H), jnp.float32),
    )(edge_w, wcat)


def _sc_body(table, sw, eidx, spi, edge_out, spb_out,
             eidx_v, idx_v, acc_v, spb_v, sem_f, sem_a, sem_s):
    wid = lax.axis_index("s") * _NC + lax.axis_index("c")
    n_chunks = edge_out.shape[0] // (_NW * _C)
    iota = lax.iota(jnp.int32, 16)

    def sub(k, carry):
        base = (wid * n_chunks + k) * _C
        pltpu.sync_copy(eidx.at[pl.ds(base, _C)], eidx_v)
        pltpu.sync_copy(spi.at[pl.ds(base, _C)], idx_v.at[_PAIRS])
        for p in range(_PAIRS):
            d = p // 3
            for g in range(_C // 16):
                rows = g * 16 + iota
                cols = jnp.full((16,), p, jnp.int32)
                v = plsc.load_gather(eidx_v, [rows, cols])
                idx_v[p, pl.ds(g * 16, 16)] = v * _D + d
        pltpu.async_copy(table.at[idx_v.at[0]], acc_v, sem_f).wait()
        cps = [pltpu.async_copy(table.at[idx_v.at[p]], acc_v, sem_a, add=True)
               for p in range(1, _PAIRS)]
        sp_cp = pltpu.async_copy(sw.at[idx_v.at[_PAIRS]], spb_v, sem_s)
        for cp in cps:
            cp.wait()
        sp_cp.wait()
        pltpu.sync_copy(acc_v, edge_out.at[pl.ds(base, _C)])
        pltpu.sync_copy(spb_v, spb_out.at[pl.ds(base, _C)])
        return carry

    lax.fori_loop(0, n_chunks, sub, 0)


def _sc_gather(table, sw, eidx, spi, bnn):
    f = functools.partial(
        pl.kernel,
        out_type=[jax.ShapeDtypeStruct((bnn, _H), jnp.float32),
                  jax.ShapeDtypeStruct((bnn, _H), jnp.float32)],
        mesh=plsc.VectorSubcoreMesh(core_axis_name="c", subcore_axis_name="s",
                                    num_cores=_NC, num_subcores=_NS),
        scratch_types=[pltpu.VMEM((_C, _PAIRS), jnp.int32),
                       pltpu.VMEM((_PAIRS + 1, _C), jnp.int32),
                       pltpu.VMEM((_C, _H), jnp.float32),
                       pltpu.VMEM((_C, _H), jnp.float32),
                       pltpu.SemaphoreType.DMA,
                       pltpu.SemaphoreType.DMA,
                       pltpu.SemaphoreType.DMA],
        compiler_params=pltpu.CompilerParams(use_tc_tiling_on_sc=False,
                                             needs_layout_passes=False),
    )(_sc_body)
    return f(table, sw, eidx, spi)


def _asm_body(ab_ref, sp_ref, edge_ref, spb_ref, t_ref, o_ref):
    n = 64
    sp = sp_ref[0]
    spc = jnp.where(sp == 0, 1, sp)
    spc = jnp.where(spc > 1, spc - 1, spc)
    spc = jnp.clip(spc, 0, _D)
    rs = 1.0 / spc.astype(jnp.float32)
    e_t = jnp.transpose(edge_ref[...])
    s_t = jnp.transpose(spb_ref[...])
    interior = e_t * rs + s_t + 2.0 * ab_ref[0]
    interior = interior.reshape(_H, n, n)
    full = jnp.pad(interior, ((0, 0), (1, 0), (1, 0)))
    ii = lax.broadcasted_iota(jnp.int32, (n + 1, n + 1), 0)
    jj = lax.broadcasted_iota(jnp.int32, (n + 1, n + 1), 1)
    border = (ii == 0) ^ (jj == 0)
    t = t_ref[...]
    full = full + jnp.where(border[None], t[:, :, None], 0.0)
    o_ref[0] = full


def _assemble(ab2, sp2, edge_acc, spb, t_col):
    b, _, nn = ab2.shape
    n1 = 65
    return pl.pallas_call(
        _asm_body,
        grid=(b,),
        in_specs=[
            pl.BlockSpec((1, 1, nn), lambda i: (i, 0, 0)),
            pl.BlockSpec((1, 1, nn), lambda i: (i, 0, 0)),
            pl.BlockSpec((nn, _H), lambda i: (i, 0)),
            pl.BlockSpec((nn, _H), lambda i: (i, 0)),
            pl.BlockSpec((_H, 1), lambda i: (0, 0)),
        ],
        out_specs=pl.BlockSpec((1, _H, n1, n1), lambda i: (i, 0, 0, 0)),
        out_shape=jax.ShapeDtypeStruct((b, _H, n1, n1), jnp.float32),
    )(ab2, sp2, edge_acc, spb, t_col)


def kernel(attn_bias, spatial_pos, x, edge_input, attn_edge_type,
           edge_encoder_w, spatial_pos_encoder_w, edge_dis_encoder_w,
           graph_token_virtual_distance_w):
    b, n = x.shape[0], x.shape[1]
    bnn = b * n * n

    w = edge_dis_encoder_w.reshape(-1, _H, _H)[:_D]
    wcat = jnp.transpose(w, (1, 0, 2)).reshape(_H, _D * _H)
    ct = _build_table(edge_encoder_w, wcat)
    table = ct.reshape(_EROWS, _H)

    eidx = edge_input.astype(jnp.int32).reshape(bnn, _PAIRS)
    spi = spatial_pos.astype(jnp.int32).reshape(bnn)

    edge_acc, spb = _sc_gather(table, spatial_pos_encoder_w, eidx, spi, bnn)

    return _assemble(
        attn_bias.reshape(b, 1, n * n),
        spatial_pos.astype(jnp.int32).reshape(b, 1, n * n),
        edge_acc, spb,
        graph_token_virtual_distance_w.reshape(_H, 1),
    )


# speedup vs baseline: 1.4067x; 1.4067x over previous
import functools

import jax
import jax.numpy as jnp
from jax import lax
from jax.experimental import pallas as pl
from jax.experimental.pallas import tpu as pltpu
from jax.experimental.pallas import tpu_sc as plsc

_H = 32
_D = 5
_PAIRS = _D * 3
_EROWS = 1537 * _D
_SROWS = 512
_TROWS = _EROWS + _SROWS

_NC, _NS = 2, 16
_NW = _NC * _NS
_C = 128


def _table_body(e_ref, w_ref, o_ref):
    o_ref[...] = (jnp.dot(e_ref[...], w_ref[...],
                          preferred_element_type=jnp.float32)
                  * (1.0 / 3.0)).astype(jnp.bfloat16)


def _build_table(edge_w, wcat):
    return pl.pallas_call(
        _table_body,
        out_shape=jax.ShapeDtypeStruct((edge_w.shape[0], _D * _H),
                                       jnp.bfloat16),
    )(edge_w, wcat)


def _sc_body(table, idx, edge_out, spb_out, idx_v, acc_v, spb_v,
             sem_a, sem_s):
    wid = lax.axis_index("s") * _NC + lax.axis_index("c")
    n_chunks = edge_out.shape[0] // (_NW * _C)
    zrow = jnp.zeros((2 * 16,), jnp.bfloat16)

    def sub(k, carry):
        base = (wid * n_chunks + k) * _C
        pltpu.sync_copy(idx.at[:, pl.ds(base, _C)], idx_v)
        for r in range(_C):
            acc_v[r] = zrow
        cps = [pltpu.async_copy(table.at[idx_v.at[p]], acc_v, sem_a, add=True)
               for p in range(_PAIRS)]
        sp_cp = pltpu.async_copy(table.at[idx_v.at[_PAIRS]], spb_v, sem_s)
        for cp in cps:
            cp.wait()
        sp_cp.wait()
        pltpu.sync_copy(acc_v, edge_out.at[pl.ds(base, _C)])
        pltpu.sync_copy(spb_v, spb_out.at[pl.ds(base, _C)])
        return carry

    lax.fori_loop(0, n_chunks, sub, 0)


def _sc_gather(table, idx, bnn):
    f = functools.partial(
        pl.kernel,
        out_type=[jax.ShapeDtypeStruct((bnn, _H), jnp.bfloat16),
                  jax.ShapeDtypeStruct((bnn, _H), jnp.bfloat16)],
        mesh=plsc.VectorSubcoreMesh(core_axis_name="c", subcore_axis_name="s",
                                    num_cores=_NC, num_subcores=_NS),
        scratch_types=[pltpu.VMEM((_PAIRS + 1, _C), jnp.int32),
                       pltpu.VMEM((_C, _H), jnp.bfloat16),
                       pltpu.VMEM((_C, _H), jnp.bfloat16),
                       pltpu.SemaphoreType.DMA,
                       pltpu.SemaphoreType.DMA],
        compiler_params=pltpu.CompilerParams(use_tc_tiling_on_sc=False,
                                             needs_layout_passes=False),
    )(_sc_body)
    return f(table, idx)


def _asm_body(ab_ref, sp_ref, edge_ref, spb_ref, t_ref, o_ref):
    n = 64
    sp = sp_ref[0]
    spc = jnp.where(sp == 0, 1, sp)
    spc = jnp.where(spc > 1, spc - 1, spc)
    spc = jnp.clip(spc, 0, _D)
    rs = 1.0 / spc.astype(jnp.float32)
    e_t = jnp.transpose(edge_ref[...]).astype(jnp.float32)
    s_t = jnp.transpose(spb_ref[...]).astype(jnp.float32)
    interior = e_t * rs + s_t + 2.0 * ab_ref[0]
    interior = interior.reshape(_H, n, n)
    full = jnp.pad(interior, ((0, 0), (1, 0), (1, 0)))
    ii = lax.broadcasted_iota(jnp.int32, (n + 1, n + 1), 0)
    jj = lax.broadcasted_iota(jnp.int32, (n + 1, n + 1), 1)
    border = (ii == 0) ^ (jj == 0)
    t = t_ref[...]
    full = full + jnp.where(border[None], t[:, :, None], 0.0)
    o_ref[0] = full


def _assemble(ab2, sp2, edge_acc, spb, t_col):
    b, _, nn = ab2.shape
    n1 = 65
    return pl.pallas_call(
        _asm_body,
        grid=(b,),
        in_specs=[
            pl.BlockSpec((1, 1, nn), lambda i: (i, 0, 0)),
            pl.BlockSpec((1, 1, nn), lambda i: (i, 0, 0)),
            pl.BlockSpec((nn, _H), lambda i: (i, 0)),
            pl.BlockSpec((nn, _H), lambda i: (i, 0)),
            pl.BlockSpec((_H, 1), lambda i: (0, 0)),
        ],
        out_specs=pl.BlockSpec((1, _H, n1, n1), lambda i: (i, 0, 0, 0)),
        out_shape=jax.ShapeDtypeStruct((b, _H, n1, n1), jnp.float32),
    )(ab2, sp2, edge_acc, spb, t_col)


def kernel(attn_bias, spatial_pos, x, edge_input, attn_edge_type,
           edge_encoder_w, spatial_pos_encoder_w, edge_dis_encoder_w,
           graph_token_virtual_distance_w):
    b, n = x.shape[0], x.shape[1]
    bnn = b * n * n

    w = edge_dis_encoder_w.reshape(-1, _H, _H)[:_D]
    wcat = jnp.transpose(w, (1, 0, 2)).reshape(_H, _D * _H)
    ct = _build_table(edge_encoder_w, wcat)
    table = jnp.concatenate(
        [ct.reshape(_EROWS, _H),
         spatial_pos_encoder_w.astype(jnp.bfloat16)], axis=0)

    e = edge_input.astype(jnp.int32)
    eidx = e * _D + jnp.arange(_D, dtype=jnp.int32)[:, None]
    eidx = eidx.reshape(bnn, _PAIRS)
    sidx = spatial_pos.astype(jnp.int32).reshape(bnn, 1) + _EROWS
    idx = jnp.concatenate([eidx, sidx], axis=1).T

    edge_acc, spb = _sc_gather(table, idx, bnn)

    return _assemble(
        attn_bias.reshape(b, 1, n * n),
        spatial_pos.astype(jnp.int32).reshape(b, 1, n * n),
        edge_acc, spb,
        graph_token_virtual_distance_w.reshape(_H, 1),
    )
